# baseline (speedup 1.0000x reference)
import functools

import jax
import jax.numpy as jnp
from jax import lax
from jax.experimental import pallas as pl
from jax.experimental.pallas import tpu as pltpu
from jax.experimental.pallas import tpu_sc as plsc

_PX = 640
_PY = 640
_P = _PX * _PY
_C = 64
_EPS = 1e-5
_NPAD = 102400
_WIN = 2048
_NWIN = _NPAD // _WIN
_R = 4
_CELLS = _P // _R
_NG = 4
_NWORK = 32
_UNITS_PER_W = _NG * _C * _R // _NWORK


def _stats_body(x_ref, o_ref):
    g = pl.program_id(0)
    wi = pl.program_id(1)
    blk = x_ref[0]
    m = blk[3:4]
    v = blk[0:3]
    q = jnp.concatenate([
        v * m,
        v[0:1] * v * m,
        v[1:2] * v[1:3] * m,
        v[2:3] * v[2:3] * m,
        m,
    ], axis=0)
    fsel = (g % 2).astype(jnp.float32)
    z2 = jnp.zeros((2, _WIN), jnp.float32)
    q24 = jnp.concatenate([q * (1.0 - fsel), z2, q * fsel, z2], axis=0)
    part = q24.reshape(24, _WIN // 128, 128).sum(axis=1)

    @pl.when(jnp.logical_and(g == 0, wi == 0))
    def _():
        o_ref[...] = part

    @pl.when(jnp.logical_not(jnp.logical_and(g == 0, wi == 0)))
    def _():
        o_ref[...] += part


def _run_stats(xpk):
    return pl.pallas_call(
        _stats_body,
        grid=(_NG, _NWIN),
        in_specs=[pl.BlockSpec((1, 8, _WIN), lambda g, w: (g, 0, w))],
        out_specs=pl.BlockSpec((24, 128), lambda g, w: (0, 0)),
        out_shape=jax.ShapeDtypeStruct((24, 128), jnp.float32),
    )(xpk)


def _emb_body(x_ref, w_ref, b_ref, o_ref):
    blk = x_ref[0]
    w = w_ref[0]
    bb = b_ref[0][:, 0:1]
    m = blk[3:4]
    h = lax.dot_general(w, blk, (((1,), (0,)), ((), ())),
                        preferred_element_type=jnp.float32)
    o_ref[0] = jnp.maximum(h + bb, 0.0) * m


def _run_emb(xpk, w2p, b2p):
    return pl.pallas_call(
        _emb_body,
        grid=(_NG, _NWIN),
        in_specs=[
            pl.BlockSpec((1, 8, _WIN), lambda g, w: (g, 0, w)),
            pl.BlockSpec((1, _C, 8), lambda g, w: (g, 0, 0)),
            pl.BlockSpec((1, _C, 128), lambda g, w: (g, 0, 0)),
        ],
        out_specs=pl.BlockSpec((1, _C, _WIN), lambda g, w: (g, 0, w)),
        out_shape=jax.ShapeDtypeStruct((_NG, _C, _NPAD), jnp.float32),
    )(xpk, w2p, b2p)


def _sc_scatter_body(emb_hbm, idx_hbm, out_hbm, dense, idxb, embb):
    cid = lax.axis_index("c")
    sid = lax.axis_index("s")
    w = sid * 2 + cid

    def unit_body(j, _):
        u = w * _UNITS_PER_W + j
        g = u // (_C * _R)
        rem = u % (_C * _R)
        ch = rem // _R
        r = rem % _R
        lo = r * _CELLS

        def zero_body(k, _):
            dense[pl.ds(k * 16, 16)] = jnp.zeros((16,), jnp.float32)
            return 0

        lax.fori_loop(0, _CELLS // 16, zero_body, 0)

        def win_body(wi, _):
            pltpu.sync_copy(idx_hbm.at[g, pl.ds(wi * _WIN, _WIN)], idxb)
            pltpu.sync_copy(emb_hbm.at[g, ch, pl.ds(wi * _WIN, _WIN)], embb)

            def vec_body(k, _):
                iv = idxb[pl.ds(k * 16, 16)]
                ev = embb[pl.ds(k * 16, 16)]
                msk = jnp.logical_and(iv >= lo, iv < lo + _CELLS)
                local = jnp.where(msk, iv - lo, 0)
                plsc.addupdate_scatter(dense, [local], ev, mask=msk)
                return 0

            lax.fori_loop(0, _WIN // 16, vec_body, 0)
            return 0

        lax.fori_loop(0, _NWIN, win_body, 0)
        pltpu.sync_copy(dense, out_hbm.at[g, ch, pl.ds(lo, _CELLS)])
        return 0

    lax.fori_loop(0, _UNITS_PER_W, unit_body, 0)


@functools.cache
def _sc_scatter():
    return pl.kernel(
        _sc_scatter_body,
        out_type=jax.ShapeDtypeStruct((_NG, _C, _P), jnp.float32),
        mesh=plsc.VectorSubcoreMesh(core_axis_name="c", subcore_axis_name="s"),
        compiler_params=pltpu.CompilerParams(needs_layout_passes=False),
        scratch_types=[
            pltpu.VMEM((_CELLS,), jnp.float32),
            pltpu.VMEM((_WIN,), jnp.int32),
            pltpu.VMEM((_WIN,), jnp.float32),
        ],
    )


def _fold_bn(sums, W, b, gamma, beta):
    sx = sums[0:3]
    M = jnp.stack([
        jnp.stack([sums[3], sums[4], sums[5]]),
        jnp.stack([sums[4], sums[6], sums[7]]),
        jnp.stack([sums[5], sums[7], sums[8]]),
    ])
    sm = sums[9]
    cnt = jnp.maximum(sm, 1.0)
    mh = (W @ sx + b * sm) / cnt
    eh2 = (jnp.einsum('ci,ij,cj->c', W, M, W)
           + 2.0 * b * (W @ sx) + b * b * sm) / cnt
    var = eh2 - mh * mh * (2.0 - sm / cnt)
    scale = gamma / jnp.sqrt(var + _EPS)
    W2 = W * scale[:, None]
    b2 = (b - mh) * scale + beta
    return W2, b2


def kernel(previous_pcl, previous_mask, previous_grid,
           current_pcl, current_mask, current_grid, W, b, gamma, beta):
    B, N, _ = previous_pcl.shape

    def pack(pcl, msk):
        xyz = jnp.transpose(pcl, (0, 2, 1))
        mrow = msk.astype(jnp.float32)[:, None, :]
        rows = jnp.concatenate(
            [xyz, mrow, jnp.zeros((B, 4, N), jnp.float32)], axis=1)
        return jnp.pad(rows, ((0, 0), (0, 0), (0, _NPAD - N)))

    pp = pack(previous_pcl, previous_mask)
    cp = pack(current_pcl, current_mask)
    xpk = jnp.stack([pp[0], cp[0], pp[1], cp[1]])

    pg = jnp.pad(previous_grid, ((0, 0), (0, _NPAD - N)))
    cg = jnp.pad(current_grid, ((0, 0), (0, _NPAD - N)))
    idx4 = jnp.stack([pg[0], cg[0], pg[1], cg[1]])

    stats = _run_stats(xpk).sum(axis=1)
    W2_prev, b2_prev = _fold_bn(stats[0:12], W, b, gamma, beta)
    W2_curr, b2_curr = _fold_bn(stats[12:24], W, b, gamma, beta)
    W2 = jnp.stack([W2_prev, W2_curr, W2_prev, W2_curr])
    b2 = jnp.stack([b2_prev, b2_curr, b2_prev, b2_curr])
    w2p = jnp.pad(W2, ((0, 0), (0, 0), (0, 5)))
    b2p = jnp.broadcast_to(b2[:, :, None], (_NG, _C, 128)) + 0.0

    emb_t = _run_emb(xpk, w2p, b2p)
    out = _sc_scatter()(emb_t, idx4)
    return out.reshape(2 * B, _C, _PX, _PY)


# speedup vs baseline: 2.0306x; 2.0306x over previous
import functools

import jax
import jax.numpy as jnp
from jax import lax
from jax.experimental import pallas as pl
from jax.experimental.pallas import tpu as pltpu
from jax.experimental.pallas import tpu_sc as plsc

_PX = 640
_PY = 640
_P = _PX * _PY
_C = 64
_EPS = 1e-5
_NPAD = 102400
_WIN = 2048
_NWIN = _NPAD // _WIN
_R = 4
_CELLS = _P // _R
_NG = 4
_NWORK = 32
_UNITS_PER_W = _NG * _C * _R // _NWORK


def _stats_body(x_ref, o_ref):
    g = pl.program_id(0)
    wi = pl.program_id(1)
    blk = x_ref[0]
    m = blk[3:4]
    v = blk[0:3]
    q = jnp.concatenate([
        v * m,
        v[0:1] * v * m,
        v[1:2] * v[1:3] * m,
        v[2:3] * v[2:3] * m,
        m,
    ], axis=0)
    fsel = (g % 2).astype(jnp.float32)
    z2 = jnp.zeros((2, _WIN), jnp.float32)
    q24 = jnp.concatenate([q * (1.0 - fsel), z2, q * fsel, z2], axis=0)
    part = q24.reshape(24, _WIN // 128, 128).sum(axis=1)

    @pl.when(jnp.logical_and(g == 0, wi == 0))
    def _():
        o_ref[...] = part

    @pl.when(jnp.logical_not(jnp.logical_and(g == 0, wi == 0)))
    def _():
        o_ref[...] += part


def _run_stats(xpk):
    return pl.pallas_call(
        _stats_body,
        grid=(_NG, _NWIN),
        in_specs=[pl.BlockSpec((1, 8, _WIN), lambda g, w: (g, 0, w))],
        out_specs=pl.BlockSpec((24, 128), lambda g, w: (0, 0)),
        out_shape=jax.ShapeDtypeStruct((24, 128), jnp.float32),
    )(xpk)


def _emb_body(x_ref, w_ref, b_ref, o_ref):
    blk = x_ref[0]
    w = w_ref[0]
    bb = b_ref[0][:, 0:1]
    m = blk[3:4]
    h = lax.dot_general(w, blk, (((1,), (0,)), ((), ())),
                        preferred_element_type=jnp.float32)
    o_ref[0] = jnp.maximum(h + bb, 0.0) * m


def _run_emb(xpk, w2p, b2p):
    return pl.pallas_call(
        _emb_body,
        grid=(_NG, _NWIN),
        in_specs=[
            pl.BlockSpec((1, 8, _WIN), lambda g, w: (g, 0, w)),
            pl.BlockSpec((1, _C, 8), lambda g, w: (g, 0, 0)),
            pl.BlockSpec((1, _C, 128), lambda g, w: (g, 0, 0)),
        ],
        out_specs=pl.BlockSpec((1, _C, _WIN), lambda g, w: (g, 0, w)),
        out_shape=jax.ShapeDtypeStruct((_NG, _C, _NPAD), jnp.float32),
    )(xpk, w2p, b2p)


def _sc_scatter_body(emb_hbm, idx_hbm, out_hbm, dense, idxb, embb,
                     sem_i0, sem_i1, sem_e0, sem_e1):
    cid = lax.axis_index("c")
    sid = lax.axis_index("s")
    w = sid * 2 + cid
    sems_i = (sem_i0, sem_i1)
    sems_e = (sem_e0, sem_e1)

    def unit_body(j, _):
        u = w * _UNITS_PER_W + j
        g = u // (_C * _R)
        rem = u % (_C * _R)
        ch = rem // _R
        r = rem % _R
        lo = r * _CELLS

        def zero_body(k, _):
            z = jnp.zeros((16,), jnp.float32)
            for t in range(4):
                dense[pl.ds(k * 64 + t * 16, 16)] = z
            return 0

        lax.fori_loop(0, _CELLS // 64, zero_body, 0)

        def issue(wi, p):
            pltpu.async_copy(idx_hbm.at[g, pl.ds(wi * _WIN, _WIN)],
                             idxb.at[p], sems_i[p])
            pltpu.async_copy(emb_hbm.at[g, ch, pl.ds(wi * _WIN, _WIN)],
                             embb.at[p], sems_e[p])

        issue(0, 0)

        def win2_body(t, _):
            for p in (0, 1):
                wi = 2 * t + p

                @pl.when(wi + 1 < _NWIN)
                def _():
                    issue(wi + 1, 1 - p)

                pltpu.make_async_copy(idx_hbm.at[g, pl.ds(wi * _WIN, _WIN)],
                                      idxb.at[p], sems_i[p]).wait()
                pltpu.make_async_copy(emb_hbm.at[g, ch, pl.ds(wi * _WIN, _WIN)],
                                      embb.at[p], sems_e[p]).wait()

                def vec_body(k, _):
                    for t4 in range(4):
                        off = k * 64 + t4 * 16
                        iv = idxb[p, pl.ds(off, 16)]
                        ev = embb[p, pl.ds(off, 16)]
                        msk = jnp.logical_and(iv >= lo, iv < lo + _CELLS)
                        local = jnp.where(msk, iv - lo, 0)
                        plsc.addupdate_scatter(dense, [local], ev, mask=msk)
                    return 0

                lax.fori_loop(0, _WIN // 64, vec_body, 0)
            return 0

        lax.fori_loop(0, _NWIN // 2, win2_body, 0)
        pltpu.sync_copy(dense, out_hbm.at[g, ch, pl.ds(lo, _CELLS)])
        return 0

    lax.fori_loop(0, _UNITS_PER_W, unit_body, 0)


@functools.cache
def _sc_scatter():
    return pl.kernel(
        _sc_scatter_body,
        out_type=jax.ShapeDtypeStruct((_NG, _C, _P), jnp.float32),
        mesh=plsc.VectorSubcoreMesh(core_axis_name="c", subcore_axis_name="s"),
        compiler_params=pltpu.CompilerParams(needs_layout_passes=False),
        scratch_types=[
            pltpu.VMEM((_CELLS,), jnp.float32),
            pltpu.VMEM((2, _WIN), jnp.int32),
            pltpu.VMEM((2, _WIN), jnp.float32),
            pltpu.SemaphoreType.DMA,
            pltpu.SemaphoreType.DMA,
            pltpu.SemaphoreType.DMA,
            pltpu.SemaphoreType.DMA,
        ],
    )


def _fold_bn(sums, W, b, gamma, beta):
    sx = sums[0:3]
    M = jnp.stack([
        jnp.stack([sums[3], sums[4], sums[5]]),
        jnp.stack([sums[4], sums[6], sums[7]]),
        jnp.stack([sums[5], sums[7], sums[8]]),
    ])
    sm = sums[9]
    cnt = jnp.maximum(sm, 1.0)
    mh = (W @ sx + b * sm) / cnt
    eh2 = (jnp.einsum('ci,ij,cj->c', W, M, W)
           + 2.0 * b * (W @ sx) + b * b * sm) / cnt
    var = eh2 - mh * mh * (2.0 - sm / cnt)
    scale = gamma / jnp.sqrt(var + _EPS)
    W2 = W * scale[:, None]
    b2 = (b - mh) * scale + beta
    return W2, b2


def kernel(previous_pcl, previous_mask, previous_grid,
           current_pcl, current_mask, current_grid, W, b, gamma, beta):
    B, N, _ = previous_pcl.shape

    def pack(pcl, msk):
        xyz = jnp.transpose(pcl, (0, 2, 1))
        mrow = msk.astype(jnp.float32)[:, None, :]
        rows = jnp.concatenate(
            [xyz, mrow, jnp.zeros((B, 4, N), jnp.float32)], axis=1)
        return jnp.pad(rows, ((0, 0), (0, 0), (0, _NPAD - N)))

    pp = pack(previous_pcl, previous_mask)
    cp = pack(current_pcl, current_mask)
    xpk = jnp.stack([pp[0], cp[0], pp[1], cp[1]])

    pg = jnp.pad(previous_grid, ((0, 0), (0, _NPAD - N)))
    cg = jnp.pad(current_grid, ((0, 0), (0, _NPAD - N)))
    idx4 = jnp.stack([pg[0], cg[0], pg[1], cg[1]])

    stats = _run_stats(xpk).sum(axis=1)
    W2_prev, b2_prev = _fold_bn(stats[0:12], W, b, gamma, beta)
    W2_curr, b2_curr = _fold_bn(stats[12:24], W, b, gamma, beta)
    W2 = jnp.stack([W2_prev, W2_curr, W2_prev, W2_curr])
    b2 = jnp.stack([b2_prev, b2_curr, b2_prev, b2_curr])
    w2p = jnp.pad(W2, ((0, 0), (0, 0), (0, 5)))
    b2p = jnp.broadcast_to(b2[:, :, None], (_NG, _C, 128)) + 0.0

    emb_t = _run_emb(xpk, w2p, b2p)
    out = _sc_scatter()(emb_t, idx4)
    return out.reshape(2 * B, _C, _PX, _PY)


# speedup vs baseline: 2.1141x; 1.0411x over previous
import functools

import jax
import jax.numpy as jnp
from jax import lax
from jax.experimental import pallas as pl
from jax.experimental.pallas import tpu as pltpu
from jax.experimental.pallas import tpu_sc as plsc

_PX = 640
_PY = 640
_P = _PX * _PY
_C = 64
_EPS = 1e-5
_NPAD = 102400
_WIN = 2048
_NWIN = _NPAD // _WIN
_R = 4
_CELLS = _P // _R
_NG = 4
_NWORK = 32
_UNITS_PER_W = _NG * _C * _R // _NWORK


def _stats_body(x_ref, o_ref):
    g = pl.program_id(0)
    wi = pl.program_id(1)
    blk = x_ref[0]
    m = blk[3:4]
    v = blk[0:3]
    q = jnp.concatenate([
        v * m,
        v[0:1] * v * m,
        v[1:2] * v[1:3] * m,
        v[2:3] * v[2:3] * m,
        m,
    ], axis=0)
    fsel = (g % 2).astype(jnp.float32)
    z2 = jnp.zeros((2, _WIN), jnp.float32)
    q24 = jnp.concatenate([q * (1.0 - fsel), z2, q * fsel, z2], axis=0)
    part = q24.reshape(24, _WIN // 128, 128).sum(axis=1)

    @pl.when(jnp.logical_and(g == 0, wi == 0))
    def _():
        o_ref[...] = part

    @pl.when(jnp.logical_not(jnp.logical_and(g == 0, wi == 0)))
    def _():
        o_ref[...] += part


def _run_stats(xpk):
    return pl.pallas_call(
        _stats_body,
        grid=(_NG, _NWIN),
        in_specs=[pl.BlockSpec((1, 8, _WIN), lambda g, w: (g, 0, w))],
        out_specs=pl.BlockSpec((24, 128), lambda g, w: (0, 0)),
        out_shape=jax.ShapeDtypeStruct((24, 128), jnp.float32),
    )(xpk)


def _emb_body(x_ref, w_ref, b_ref, o_ref):
    blk = x_ref[0]
    w = w_ref[0]
    bb = b_ref[0][:, 0:1]
    m = blk[3:4]
    h = lax.dot_general(w, blk, (((1,), (0,)), ((), ())),
                        preferred_element_type=jnp.float32)
    o_ref[0] = jnp.maximum(h + bb, 0.0) * m


def _run_emb(xpk, w2p, b2p):
    return pl.pallas_call(
        _emb_body,
        grid=(_NG, _NWIN),
        in_specs=[
            pl.BlockSpec((1, 8, _WIN), lambda g, w: (g, 0, w)),
            pl.BlockSpec((1, _C, 8), lambda g, w: (g, 0, 0)),
            pl.BlockSpec((1, _C, 128), lambda g, w: (g, 0, 0)),
        ],
        out_specs=pl.BlockSpec((1, _C, _WIN), lambda g, w: (g, 0, w)),
        out_shape=jax.ShapeDtypeStruct((_NG, _C, _NPAD), jnp.float32),
    )(xpk, w2p, b2p)


def _sc_scatter_body(emb_hbm, idx_hbm, out_hbm, dense, idxb, embb,
                     sem_i0, sem_i1, sem_e0, sem_e1):
    cid = lax.axis_index("c")
    sid = lax.axis_index("s")
    w = sid * 2 + cid
    sems_i = (sem_i0, sem_i1)
    sems_e = (sem_e0, sem_e1)

    def unit_body(j, _):
        u = w * _UNITS_PER_W + j
        g = u // (_C * _R)
        rem = u % (_C * _R)
        ch = rem // _R
        r = rem % _R
        lo = r * _CELLS

        def zero_body(k, _):
            z = jnp.zeros((16,), jnp.float32)
            for t in range(4):
                dense[pl.ds(k * 64 + t * 16, 16)] = z
            return 0

        lax.fori_loop(0, _CELLS // 64, zero_body, 0)

        def issue(wi, p):
            pltpu.async_copy(idx_hbm.at[g, pl.ds(wi * _WIN, _WIN)],
                             idxb.at[p], sems_i[p])
            pltpu.async_copy(emb_hbm.at[g, ch, pl.ds(wi * _WIN, _WIN)],
                             embb.at[p], sems_e[p])

        issue(0, 0)

        def win2_body(t, _):
            for p in (0, 1):
                wi = 2 * t + p

                @pl.when(wi + 1 < _NWIN)
                def _():
                    issue(wi + 1, 1 - p)

                pltpu.make_async_copy(idx_hbm.at[g, pl.ds(wi * _WIN, _WIN)],
                                      idxb.at[p], sems_i[p]).wait()
                pltpu.make_async_copy(emb_hbm.at[g, ch, pl.ds(wi * _WIN, _WIN)],
                                      embb.at[p], sems_e[p]).wait()

                def vec_body(k, _):
                    for t4 in range(8):
                        off = k * 128 + t4 * 16
                        iv = idxb[p, pl.ds(off, 16)]
                        ev = embb[p, pl.ds(off, 16)]
                        msk = jnp.logical_and(iv >= lo, iv < lo + _CELLS)
                        local = jnp.where(msk, iv - lo, 0)
                        plsc.addupdate_scatter(dense, [local], ev, mask=msk)
                    return 0

                lax.fori_loop(0, _WIN // 128, vec_body, 0)
            return 0

        lax.fori_loop(0, _NWIN // 2, win2_body, 0)
        pltpu.sync_copy(dense, out_hbm.at[g, ch, pl.ds(lo, _CELLS)])
        return 0

    lax.fori_loop(0, _UNITS_PER_W, unit_body, 0)


@functools.cache
def _sc_scatter():
    return pl.kernel(
        _sc_scatter_body,
        out_type=jax.ShapeDtypeStruct((_NG, _C, _P), jnp.float32),
        mesh=plsc.VectorSubcoreMesh(core_axis_name="c", subcore_axis_name="s"),
        compiler_params=pltpu.CompilerParams(needs_layout_passes=False),
        scratch_types=[
            pltpu.VMEM((_CELLS,), jnp.float32),
            pltpu.VMEM((2, _WIN), jnp.int32),
            pltpu.VMEM((2, _WIN), jnp.float32),
            pltpu.SemaphoreType.DMA,
            pltpu.SemaphoreType.DMA,
            pltpu.SemaphoreType.DMA,
            pltpu.SemaphoreType.DMA,
        ],
    )


def _fold_bn(sums, W, b, gamma, beta):
    sx = sums[0:3]
    M = jnp.stack([
        jnp.stack([sums[3], sums[4], sums[5]]),
        jnp.stack([sums[4], sums[6], sums[7]]),
        jnp.stack([sums[5], sums[7], sums[8]]),
    ])
    sm = sums[9]
    cnt = jnp.maximum(sm, 1.0)
    mh = (W @ sx + b * sm) / cnt
    eh2 = (jnp.einsum('ci,ij,cj->c', W, M, W)
           + 2.0 * b * (W @ sx) + b * b * sm) / cnt
    var = eh2 - mh * mh * (2.0 - sm / cnt)
    scale = gamma / jnp.sqrt(var + _EPS)
    W2 = W * scale[:, None]
    b2 = (b - mh) * scale + beta
    return W2, b2


def kernel(previous_pcl, previous_mask, previous_grid,
           current_pcl, current_mask, current_grid, W, b, gamma, beta):
    B, N, _ = previous_pcl.shape

    def pack(pcl, msk):
        xyz = jnp.transpose(pcl, (0, 2, 1))
        mrow = msk.astype(jnp.float32)[:, None, :]
        rows = jnp.concatenate(
            [xyz, mrow, jnp.zeros((B, 4, N), jnp.float32)], axis=1)
        return jnp.pad(rows, ((0, 0), (0, 0), (0, _NPAD - N)))

    pp = pack(previous_pcl, previous_mask)
    cp = pack(current_pcl, current_mask)
    xpk = jnp.stack([pp[0], cp[0], pp[1], cp[1]])

    pg = jnp.pad(previous_grid, ((0, 0), (0, _NPAD - N)))
    cg = jnp.pad(current_grid, ((0, 0), (0, _NPAD - N)))
    idx4 = jnp.stack([pg[0], cg[0], pg[1], cg[1]])

    stats = _run_stats(xpk).sum(axis=1)
    W2_prev, b2_prev = _fold_bn(stats[0:12], W, b, gamma, beta)
    W2_curr, b2_curr = _fold_bn(stats[12:24], W, b, gamma, beta)
    W2 = jnp.stack([W2_prev, W2_curr, W2_prev, W2_curr])
    b2 = jnp.stack([b2_prev, b2_curr, b2_prev, b2_curr])
    w2p = jnp.pad(W2, ((0, 0), (0, 0), (0, 5)))
    b2p = jnp.broadcast_to(b2[:, :, None], (_NG, _C, 128)) + 0.0

    emb_t = _run_emb(xpk, w2p, b2p)
    out = _sc_scatter()(emb_t, idx4)
    return out.reshape(2 * B, _C, _PX, _PY)
